# baseline (speedup 1.0000x reference)
import jax
import jax.numpy as jnp
from jax import lax
from jax.experimental import pallas as pl
from jax.experimental.pallas import tpu as pltpu
from jax.experimental.pallas import tpu_sc as plsc

N = 10000
E = 160000
B = 16
NPAD = 10240
NT = 320
NTILES = 32


def _lanebc(vec, l):
    return vec.at[jnp.full((16,), l, jnp.int32)].get(mode="promise_in_bounds")


def _extract(vec, l, iota):
    return jnp.sum(jnp.where(iota == l, vec, 0))


def _make_stats_kernel(dp, ch, npad, nt, fix_max):
    mesh = plsc.VectorSubcoreMesh(core_axis_name="c", subcore_axis_name="s")

    def body(xp, toffs, srcs, dsts, out_s, out_q, out_m, out_c,
             offv, idx_v, dst_v, rows_v, idx_v1, dst_v1, rows_v1,
             accs, accq, accm, accc, sem, sem1):
        cid = lax.axis_index("c")
        sid = lax.axis_index("s")
        t = sid * 2 + cid
        nlo = t * nt
        iota = lax.iota(jnp.int32, 16)
        pltpu.sync_copy(toffs.at[t], offv)
        ov = offv[...]
        a0 = _extract(ov, 0, iota)
        s0 = _extract(ov, 1, iota)
        e0 = _extract(ov, 2, iota)

        zf = jnp.zeros((16,), jnp.float32)
        nf = jnp.full((16,), -3.4e38, jnp.float32)

        def init_body(i, _):
            accs[pl.ds(i * 16, 16)] = zf
            accq[pl.ds(i * 16, 16)] = zf
            accm[pl.ds(i * 16, 16)] = nf
            return 0
        lax.fori_loop(0, nt * dp // 16, init_body, 0)

        def initc(i, _):
            accc[pl.ds(i * 16, 16)] = zf
            return 0
        lax.fori_loop(0, nt // 16, initc, 0)

        nchunks = (e0 - a0 + ch - 1) // ch
        npairs = (nchunks + 1) // 2
        bufs = ((idx_v, dst_v, rows_v, sem), (idx_v1, dst_v1, rows_v1, sem1))

        def issue(b, base):
            iv, dv, rv, sm = bufs[b]
            pltpu.sync_copy(srcs.at[pl.ds(base, ch)], iv)
            pltpu.sync_copy(dsts.at[pl.ds(base, ch)], dv)
            return pltpu.async_copy(xp.at[iv], rv, sm)

        def process(b, base):
            iv, dv, rows_b, sm = bufs[b]
            for j in range(ch // 16):
                dst16 = dv[pl.ds(16 * j, 16)]
                ge = base + 16 * j + iota
                vi = jnp.where(jnp.logical_and(ge >= s0, ge < e0), 1, 0)
                for l in range(16):
                    dsel = _lanebc(dst16, l)
                    vb = _lanebc(vi, l) != 0
                    rowb = (dsel - nlo) * dp
                    rsplat = jnp.full((16,), 16 * j + l, jnp.int32)
                    for g in range(dp // 16):
                        cols = 16 * g + iota
                        msg = plsc.load_gather(rows_b, [rsplat, cols])
                        addr = rowb + cols
                        plsc.addupdate_scatter(accs, [addr], msg, mask=vb)
                        plsc.addupdate_scatter(accq, [addr], msg * msg, mask=vb)
                        cur = plsc.load_gather(accm, [addr], mask=vb)
                        plsc.store_scatter(accm, [addr], jnp.maximum(cur, msg), mask=vb)
                    plsc.addupdate_scatter(
                        accc, [dsel - nlo], jnp.full((16,), 1.0, jnp.float32),
                        mask=jnp.logical_and(vb, iota == 0))

        def pair_body(cp, _):
            b0 = pl.multiple_of(a0 + (2 * cp) * ch, 8)
            b1 = pl.multiple_of(b0 + ch, 8)
            d0 = issue(0, b0)
            d1 = issue(1, b1)
            d0.wait()
            process(0, b0)
            d1.wait()
            process(1, b1)
            return 0
        lax.fori_loop(0, npairs, pair_body, 0)

        if fix_max:
            for i in range(nt * dp // 16):
                nid = (i * 16 + iota) // dp
                cvals = plsc.load_gather(accc, [nid])
                blk = accm[pl.ds(i * 16, 16)]
                accm[pl.ds(i * 16, 16)] = jnp.where(cvals > 0, blk, 0.0)

        wb = pl.multiple_of(nlo * dp, 8)
        pltpu.sync_copy(accs, out_s.at[pl.ds(wb, nt * dp)])
        pltpu.sync_copy(accq, out_q.at[pl.ds(wb, nt * dp)])
        pltpu.sync_copy(accm, out_m.at[pl.ds(wb, nt * dp)])
        pltpu.sync_copy(accc, out_c.at[pl.ds(pl.multiple_of(nlo, 8), nt)])

    f32 = jnp.float32
    return pl.kernel(
        body,
        out_type=(
            jax.ShapeDtypeStruct((npad * dp,), f32),
            jax.ShapeDtypeStruct((npad * dp,), f32),
            jax.ShapeDtypeStruct((npad * dp,), f32),
            jax.ShapeDtypeStruct((npad,), f32),
        ),
        mesh=mesh,
        compiler_params=pltpu.CompilerParams(needs_layout_passes=False),
        scratch_types=(
            pltpu.VMEM((16,), jnp.int32),
            pltpu.VMEM((ch,), jnp.int32),
            pltpu.VMEM((ch,), jnp.int32),
            pltpu.VMEM((ch, 128), f32),
            pltpu.VMEM((ch,), jnp.int32),
            pltpu.VMEM((ch,), jnp.int32),
            pltpu.VMEM((ch, 128), f32),
            pltpu.VMEM((nt * dp,), f32),
            pltpu.VMEM((nt * dp,), f32),
            pltpu.VMEM((nt * dp,), f32),
            pltpu.VMEM((nt,), f32),
            pltpu.SemaphoreType.DMA,
            pltpu.SemaphoreType.DMA,
        ),
    )


def _make_gat_kernel(hc, hw, cdim, ch, npasses):
    nts = NT // npasses
    mesh = plsc.VectorSubcoreMesh(core_axis_name="c", subcore_axis_name="s")

    def body(hp, as8, ad8, cshift, toffs, srcs, dsts, out_acc, out_den,
             offv, cval, idx_v, dst_v, hrows, arows, idx_v1, dst_v1, hrows1,
             arows1, drows, acc, accd, sem, sem1):
        cid = lax.axis_index("c")
        sid = lax.axis_index("s")
        t = sid * 2 + cid
        iota = lax.iota(jnp.int32, 16)
        pltpu.sync_copy(toffs.at[t], offv)
        pltpu.sync_copy(cshift, cval)
        ov = offv[...]
        cv = cval[...]
        zf = jnp.zeros((16,), jnp.float32)

        def pass_body(p, _):
            a0 = _extract(ov, 3 * p, iota)
            s0 = _extract(ov, 3 * p + 1, iota)
            e0 = _extract(ov, 3 * p + 2, iota)
            nlo = t * NT + p * nts

            pltpu.sync_copy(ad8.at[pl.ds(nlo, nts)], drows)

            def init_body(i, _):
                acc[pl.ds(i * 16, 16)] = zf
                return 0
            lax.fori_loop(0, nts * hc // 16, init_body, 0)

            def initd(i, _):
                accd[pl.ds(i * 16, 16)] = zf
                return 0
            lax.fori_loop(0, nts, initd, 0)

            nchunks = (e0 - a0 + ch - 1) // ch
            npairs = (nchunks + 1) // 2
            bufs = ((idx_v, dst_v, hrows, arows, sem),
                    (idx_v1, dst_v1, hrows1, arows1, sem1))

            def issue(b, base):
                iv, dv, hr, ar, sm = bufs[b]
                pltpu.sync_copy(srcs.at[pl.ds(base, ch)], iv)
                pltpu.sync_copy(dsts.at[pl.ds(base, ch)], dv)
                da = pltpu.async_copy(hp.at[iv], hr, sm)
                db = pltpu.async_copy(as8.at[iv], ar, sm)
                return da, db

            def process(b, base):
                iv, dv, hrows_b, arows_b, sm = bufs[b]
                for j in range(ch // 16):
                    dst16 = dv[pl.ds(16 * j, 16)]
                    ge = base + 16 * j + iota
                    vi = jnp.where(jnp.logical_and(ge >= s0, ge < e0), 1, 0)
                    for l in range(16):
                        rsplat = jnp.full((16,), 16 * j + l, jnp.int32)
                        dsel = _lanebc(dst16, l)
                        vb = _lanebc(vi, l) != 0
                        av = plsc.load_gather(arows_b, [rsplat, iota])
                        adv = plsc.load_gather(drows, [dsel - nlo, iota], mask=vb)
                        ev = av + adv
                        ev = jnp.where(ev >= 0, ev, 0.2 * ev)
                        exv = jnp.exp(ev - cv)
                        plsc.addupdate_scatter(
                            accd, [(dsel - nlo) * 16 + iota], exv, mask=vb)
                        rowb = (dsel - nlo) * hc
                        for g in range(hc // 16):
                            head = (16 * g) // cdim
                            mult = _lanebc(exv, head)
                            msg = plsc.load_gather(hrows_b, [rsplat, 16 * g + iota])
                            plsc.addupdate_scatter(
                                acc, [rowb + 16 * g + iota], msg * mult, mask=vb)

            def pair_body(cp, _):
                b0 = pl.multiple_of(a0 + (2 * cp) * ch, 8)
                b1 = pl.multiple_of(b0 + ch, 8)
                da0, db0 = issue(0, b0)
                da1, db1 = issue(1, b1)
                da0.wait()
                db0.wait()
                process(0, b0)
                da1.wait()
                db1.wait()
                process(1, b1)
                return 0
            lax.fori_loop(0, npairs, pair_body, 0)

            pltpu.sync_copy(acc, out_acc.at[pl.ds(pl.multiple_of(nlo * hc, 8), nts * hc)])
            pltpu.sync_copy(accd, out_den.at[pl.ds(pl.multiple_of(nlo * 16, 8), nts * 16)])
            return 0
        lax.fori_loop(0, npasses, pass_body, 0)

    f32 = jnp.float32
    return pl.kernel(
        body,
        out_type=(
            jax.ShapeDtypeStruct((NPAD * hc,), f32),
            jax.ShapeDtypeStruct((NPAD * 16,), f32),
        ),
        mesh=mesh,
        compiler_params=pltpu.CompilerParams(needs_layout_passes=False),
        scratch_types=(
            pltpu.VMEM((16,), jnp.int32),
            pltpu.VMEM((16,), f32),
            pltpu.VMEM((ch,), jnp.int32),
            pltpu.VMEM((ch,), jnp.int32),
            pltpu.VMEM((ch, hw), f32),
            pltpu.VMEM((ch, 128), f32),
            pltpu.VMEM((ch,), jnp.int32),
            pltpu.VMEM((ch,), jnp.int32),
            pltpu.VMEM((ch, hw), f32),
            pltpu.VMEM((ch, 128), f32),
            pltpu.VMEM((nts, 16), f32),
            pltpu.VMEM((nts * hc,), f32),
            pltpu.VMEM((nts * 16,), f32),
            pltpu.SemaphoreType.DMA,
            pltpu.SemaphoreType.DMA,
        ),
    )


def _ec_mlp_block(s_ref, q_ref, m_ref, c_ref, w1_ref, b1_ref, w2_ref, b2_ref, o_ref):
    cnt = c_ref[...]
    denom = jnp.maximum(cnt, 1.0)
    s = s_ref[...]
    q = q_ref[...]
    m = jnp.where(cnt > 0, m_ref[...], 0.0)
    means = s / denom
    var = jnp.maximum(q / denom - means * means, 0.0)
    aggr = jnp.concatenate([s, m, means, var], axis=1)
    h = jnp.maximum(jnp.dot(aggr, w1_ref[...], preferred_element_type=jnp.float32) + b1_ref[...], 0.0)
    z = jnp.dot(h, w2_ref[...], preferred_element_type=jnp.float32) + b2_ref[...]
    o_ref[...] = jax.nn.sigmoid(z)


def _ec_mlp(s, q, m, cnt, w1p, b1p, w2p, b2p, dp, d2p):
    rb = 1024
    grid = (NPAD // rb,)
    d4 = 4 * dp
    return pl.pallas_call(
        _ec_mlp_block,
        grid=grid,
        in_specs=[
            pl.BlockSpec((rb, dp), lambda i: (i, 0)),
            pl.BlockSpec((rb, dp), lambda i: (i, 0)),
            pl.BlockSpec((rb, dp), lambda i: (i, 0)),
            pl.BlockSpec((rb, 1), lambda i: (i, 0)),
            pl.BlockSpec((d4, d2p), lambda i: (0, 0)),
            pl.BlockSpec((1, d2p), lambda i: (0, 0)),
            pl.BlockSpec((d2p, d2p), lambda i: (0, 0)),
            pl.BlockSpec((1, d2p), lambda i: (0, 0)),
        ],
        out_specs=pl.BlockSpec((rb, d2p), lambda i: (i, 0)),
        out_shape=jax.ShapeDtypeStruct((NPAD, d2p), jnp.float32),
    )(s, q, m, cnt, w1p, b1p, w2p, b2p)


def _gat_proj_block(x_ref, w_ref, as_ref, ad_ref, h_ref, a_ref, d_ref):
    h = jnp.dot(x_ref[...], w_ref[...], preferred_element_type=jnp.float32)
    h_ref[...] = h
    a_ref[...] = jnp.dot(h, as_ref[...], preferred_element_type=jnp.float32)
    d_ref[...] = jnp.dot(h, ad_ref[...], preferred_element_type=jnp.float32)


def _gat_proj(xp, wp, asw, adw):
    k = xp.shape[1]
    hw = wp.shape[1]
    rb = 512
    grid = (NPAD // rb,)
    return pl.pallas_call(
        _gat_proj_block,
        grid=grid,
        in_specs=[
            pl.BlockSpec((rb, k), lambda i: (i, 0)),
            pl.BlockSpec((k, hw), lambda i: (0, 0)),
            pl.BlockSpec((hw, 128), lambda i: (0, 0)),
            pl.BlockSpec((hw, 16), lambda i: (0, 0)),
        ],
        out_specs=[
            pl.BlockSpec((rb, hw), lambda i: (i, 0)),
            pl.BlockSpec((rb, 128), lambda i: (i, 0)),
            pl.BlockSpec((rb, 16), lambda i: (i, 0)),
        ],
        out_shape=[
            jax.ShapeDtypeStruct((NPAD, hw), jnp.float32),
            jax.ShapeDtypeStruct((NPAD, 128), jnp.float32),
            jax.ShapeDtypeStruct((NPAD, 16), jnp.float32),
        ],
    )(xp, wp, asw, adw)


def _gat_fin_block(acc_ref, den_ref, bias_ref, g_ref, b_ref, o_ref):
    z = acc_ref[...] / (den_ref[...] + 1e-16) + bias_ref[...]
    s = jax.nn.sigmoid(z)
    mu = jnp.mean(s, axis=0, keepdims=True)
    var = jnp.mean((s - mu) * (s - mu), axis=0, keepdims=True)
    o_ref[...] = (s - mu) * jax.lax.rsqrt(var + 1e-5) * g_ref[...] + b_ref[...]


def _gat_fin(acc, den_exp, bias, g, b):
    n, hc = acc.shape
    blk = min(hc, 128)
    return pl.pallas_call(
        _gat_fin_block,
        grid=(hc // blk,),
        in_specs=[
            pl.BlockSpec((n, blk), lambda j: (0, j)),
            pl.BlockSpec((n, blk), lambda j: (0, j)),
            pl.BlockSpec((1, blk), lambda j: (0, j)),
            pl.BlockSpec((1, blk), lambda j: (0, j)),
            pl.BlockSpec((1, blk), lambda j: (0, j)),
        ],
        out_specs=pl.BlockSpec((n, blk), lambda j: (0, j)),
        out_shape=jax.ShapeDtypeStruct((n, hc), jnp.float32),
    )(acc, den_exp, bias, g, b)


def _head_block(flat_ref, w1_ref, b1_ref, w2_ref, b2_ref, o_ref):
    f1 = jnp.dot(flat_ref[...], w1_ref[...], preferred_element_type=jnp.float32) + b1_ref[...]
    f1 = jnp.where(f1 > 0, f1, jnp.exp(jnp.minimum(f1, 0.0)) - 1.0)
    o_ref[...] = jnp.dot(f1, w2_ref[...], preferred_element_type=jnp.float32) + b2_ref[...]


def _head(flat, w1, b1, w2, b2):
    return pl.pallas_call(
        _head_block,
        out_shape=jax.ShapeDtypeStruct((flat.shape[0], w2.shape[1]), jnp.float32),
    )(flat, w1, b1, w2, b2)


def _pad_ec_w1(w1, d, dp, d2, d2p):
    w1p = jnp.zeros((4 * dp, d2p), jnp.float32)
    for k in range(4):
        w1p = w1p.at[k * dp:k * dp + d, :d2].set(w1[k * d:(k + 1) * d, :])
    return w1p


def _pad_mat(w, r, c):
    return jnp.zeros((r, c), jnp.float32).at[:w.shape[0], :w.shape[1]].set(w)


def _attn_mat(a, heads, cdim, hw, cols):
    m = jnp.zeros((hw, cols), jnp.float32)
    for h in range(heads):
        m = m.at[h * cdim:(h + 1) * cdim, h].set(a[h])
    return m


def _event_layer(xp, dp, d2, d2p, w1, b1, w2, b2, stats_fn, sort_args):
    srcs, dsts, toffs = sort_args
    s, q, m, c = stats_fn(xp, toffs, srcs, dsts)
    s = s.reshape(NPAD, dp)
    q = q.reshape(NPAD, dp)
    m = m.reshape(NPAD, dp)
    c = c.reshape(NPAD, 1)
    w1p = _pad_ec_w1(w1, w1.shape[0] // 4, dp, d2, d2p)
    b1p = _pad_mat(b1.reshape(1, -1), 1, d2p)
    w2p = _pad_mat(w2, d2p, d2p)
    b2p = _pad_mat(b2.reshape(1, -1), 1, d2p)
    return _ec_mlp(s, q, m, c, w1p, b1p, w2p, b2p, dp, d2p)


def _gat_layer(xin, w, a_s, a_d, bias, bn_g, bn_b, heads, cdim, gk, sort_args, kdim, hw):
    srcs, dsts, toffs = sort_args
    hc = heads * cdim
    xp = _pad_mat(xin, NPAD, kdim)
    wp = _pad_mat(w, kdim, hw)
    asw = _attn_mat(a_s, heads, cdim, hw, 128)
    adw = _attn_mat(a_d, heads, cdim, hw, 16)
    h, as8, ad8 = _gat_proj(xp, wp, asw, adw)
    cshift = jnp.maximum(jnp.max(as8) + jnp.max(ad8), 0.0)
    cshift16 = jnp.full((16,), cshift, jnp.float32)
    acc, den = gk(h, as8, ad8, cshift16, toffs, srcs, dsts)
    acc = acc.reshape(NPAD, hc)[:N]
    den8 = den.reshape(NPAD, 16)[:N, :heads]
    den_exp = jnp.repeat(den8, cdim, axis=1)
    return _gat_fin(acc, den_exp, bias.reshape(1, hc), bn_g.reshape(1, hc), bn_b.reshape(1, hc))


def kernel(x, edge_index, pos, batch,
           ec1_W1, ec1_b1, ec1_W2, ec1_b2,
           ec2_W1, ec2_b1, ec2_W2, ec2_b2,
           ec3_W1, ec3_b1, ec3_W2, ec3_b2,
           g0_W, g0_as, g0_ad, g0_b,
           g1_W, g1_as, g1_ad, g1_b,
           g2_W, g2_as, g2_ad, g2_b,
           bn0_g, bn0_b, bn1_g, bn1_b, bn2_g, bn2_b,
           fc1_W, fc1_b, fc2_W, fc2_b):
    src = edge_index[0].astype(jnp.int32)
    dst = edge_index[1].astype(jnp.int32)

    perm = jnp.argsort(dst)
    src_s = src[perm]
    dst_s = dst[perm]
    EPAD = E + 256
    srcs = jnp.zeros((EPAD,), jnp.int32).at[:E].set(src_s)
    dsts = jnp.full((EPAD,), N, jnp.int32).at[:E].set(dst_s)

    bounds80 = jnp.arange(129, dtype=jnp.int32) * 80
    offs80 = jnp.searchsorted(dst_s, bounds80).astype(jnp.int32)
    offs320 = offs80[::4]
    a320 = (offs320[:32] // 8) * 8
    toffs1 = jnp.zeros((NTILES, 16), jnp.int32)
    toffs1 = toffs1.at[:, 0].set(a320)
    toffs1 = toffs1.at[:, 1].set(offs320[:32])
    toffs1 = toffs1.at[:, 2].set(offs320[1:33])
    a80 = (offs80[:128] // 8) * 8
    toffs2 = jnp.zeros((NTILES, 16), jnp.int32)
    for p in range(4):
        toffs2 = toffs2.at[:, 3 * p].set(a80[p::4])
        toffs2 = toffs2.at[:, 3 * p + 1].set(offs80[:128][p::4])
        toffs2 = toffs2.at[:, 3 * p + 2].set(offs80[1:][p::4])
    sa1 = (srcs, dsts, toffs1)

    stats16 = _make_stats_kernel(16, 128, NPAD, NT, False)
    stats64 = _make_stats_kernel(64, 32, NPAD, NT, False)
    stats80 = _make_stats_kernel(80, 32, NPAD, NT, False)
    gat512 = _make_gat_kernel(512, 512, 64, 16, 4)
    gat128 = _make_gat_kernel(128, 128, 16, 32, 1)
    gat16 = _make_gat_kernel(16, 128, 16, 64, 1)

    xp1 = _pad_mat(x, N, 128)
    h1 = _event_layer(xp1, 16, 12, 16, ec1_W1, ec1_b1, ec1_W2, ec1_b2, stats16, sa1)
    xp2 = _pad_mat(jnp.concatenate([x, h1[:N, :12]], axis=1), N, 128)
    h2 = _event_layer(xp2, 64, 60, 64, ec2_W1, ec2_b1, ec2_W2, ec2_b2, stats64, sa1)
    xp3 = _pad_mat(jnp.concatenate([xp2[:, :15], h2[:N, :60]], axis=1), N, 128)
    h3 = _event_layer(xp3, 80, 300, 320, ec3_W1, ec3_b1, ec3_W2, ec3_b2, stats80, sa1)
    x4 = jnp.concatenate([xp3[:, :75], h3[:N, :300]], axis=1)

    g0 = _gat_layer(x4, g0_W, g0_as, g0_ad, g0_b, bn0_g, bn0_b, 8, 64, gat512,
                    (srcs, dsts, toffs2), 384, 512)
    g1 = _gat_layer(g0, g1_W, g1_as, g1_ad, g1_b, bn1_g, bn1_b, 8, 16, gat128,
                    sa1, 512, 128)
    g2 = _gat_layer(g1, g2_W, g2_as, g2_ad, g2_b, bn2_g, bn2_b, 1, 16, gat16,
                    sa1, 128, 128)

    vox = jnp.clip(jnp.floor(pos / 0.25).astype(jnp.int32), 0, 3)
    cid = vox[:, 0] * 16 + vox[:, 1] * 4 + vox[:, 2]
    gid = batch.astype(jnp.int32) * 64 + cid
    permg = jnp.argsort(gid).astype(jnp.int32)
    gid_s = gid[permg]
    EPG = 10240
    srcg = jnp.zeros((EPG,), jnp.int32).at[:N].set(permg)
    dstg = jnp.full((EPG,), 1024, jnp.int32).at[:N].set(gid_s)
    boundsg = jnp.arange(33, dtype=jnp.int32) * 32
    offsg = jnp.searchsorted(gid_s, boundsg).astype(jnp.int32)
    ag = (offsg[:32] // 8) * 8
    toffsg = jnp.zeros((NTILES, 16), jnp.int32)
    toffsg = toffsg.at[:, 0].set(ag)
    toffsg = toffsg.at[:, 1].set(offsg[:32])
    toffsg = toffsg.at[:, 2].set(offsg[1:33])
    statsg = _make_stats_kernel(16, 128, 1024, 32, True)
    g2p = _pad_mat(g2, N, 128)
    _, _, pm, _ = statsg(g2p, toffsg, srcg, dstg)
    pooled = pm.reshape(1024, 16)
    flat = pooled.reshape(B, 1024)
    return _head(flat, fc1_W, fc1_b, fc2_W, fc2_b)


# speedup vs baseline: 1.1214x; 1.1214x over previous
import jax
import jax.numpy as jnp
from jax import lax
from jax.experimental import pallas as pl
from jax.experimental.pallas import tpu as pltpu
from jax.experimental.pallas import tpu_sc as plsc

N = 10000
E = 160000
B = 16
NPAD = 10240
NT = 320
NTILES = 32


def _lanebc(vec, l):
    return vec.at[jnp.full((16,), l, jnp.int32)].get(mode="promise_in_bounds")


def _extract(vec, l, iota):
    return jnp.sum(jnp.where(iota == l, vec, 0))


def _make_stats_kernel(dp, ch, npad, nt, fix_max):
    mesh = plsc.VectorSubcoreMesh(core_axis_name="c", subcore_axis_name="s")

    def body(xp, toffs, srcs, dsts, out_s, out_q, out_m, out_c,
             offv, idx_v, dst_v, rows_v, idx_v1, dst_v1, rows_v1,
             accs, accq, accm, accc, sem, sem1):
        cid = lax.axis_index("c")
        sid = lax.axis_index("s")
        t = sid * 2 + cid
        nlo = t * nt
        iota = lax.iota(jnp.int32, 16)
        pltpu.sync_copy(toffs.at[t], offv)
        ov = offv[...]
        a0 = _extract(ov, 0, iota)
        s0 = _extract(ov, 1, iota)
        e0 = _extract(ov, 2, iota)

        zf = jnp.zeros((16,), jnp.float32)
        nf = jnp.full((16,), -3.4e38, jnp.float32)

        def init_body(i, _):
            accs[pl.ds(i * 16, 16)] = zf
            accq[pl.ds(i * 16, 16)] = zf
            accm[pl.ds(i * 16, 16)] = nf
            return 0
        lax.fori_loop(0, nt * dp // 16, init_body, 0)

        def initc(i, _):
            accc[pl.ds(i * 16, 16)] = zf
            return 0
        lax.fori_loop(0, nt // 16, initc, 0)

        nchunks = (e0 - a0 + ch - 1) // ch
        npairs = (nchunks + 1) // 2
        bufs = ((idx_v, dst_v, rows_v, sem), (idx_v1, dst_v1, rows_v1, sem1))

        def issue(b, base):
            iv, dv, rv, sm = bufs[b]
            pltpu.sync_copy(srcs.at[pl.ds(base, ch)], iv)
            pltpu.sync_copy(dsts.at[pl.ds(base, ch)], dv)
            return pltpu.async_copy(xp.at[iv], rv, sm)

        def process(b, base):
            iv, dv, rows_b, sm = bufs[b]
            for j in range(ch // 16):
                dst16 = dv[pl.ds(16 * j, 16)]
                ge = base + 16 * j + iota
                vi = jnp.where(jnp.logical_and(ge >= s0, ge < e0), 1, 0)
                for l in range(16):
                    dsel = _lanebc(dst16, l)
                    vb = _lanebc(vi, l) != 0
                    rowb = (dsel - nlo) * dp
                    for g in range(dp // 16):
                        cols = 16 * g + iota
                        msg = rows_b[16 * j + l, pl.ds(16 * g, 16)]
                        addr = rowb + cols
                        plsc.addupdate_scatter(accs, [addr], msg, mask=vb)
                        plsc.addupdate_scatter(accq, [addr], msg * msg, mask=vb)
                        cur = plsc.load_gather(accm, [addr], mask=vb)
                        plsc.store_scatter(accm, [addr], jnp.maximum(cur, msg), mask=vb)
                    plsc.addupdate_scatter(
                        accc, [dsel - nlo], jnp.full((16,), 1.0, jnp.float32),
                        mask=jnp.logical_and(vb, iota == 0))

        def pair_body(cp, _):
            b0 = pl.multiple_of(a0 + (2 * cp) * ch, 8)
            b1 = pl.multiple_of(b0 + ch, 8)
            d0 = issue(0, b0)
            d1 = issue(1, b1)
            d0.wait()
            process(0, b0)
            d1.wait()
            process(1, b1)
            return 0
        lax.fori_loop(0, npairs, pair_body, 0)

        if fix_max:
            for i in range(nt * dp // 16):
                nid = (i * 16 + iota) // dp
                cvals = plsc.load_gather(accc, [nid])
                blk = accm[pl.ds(i * 16, 16)]
                accm[pl.ds(i * 16, 16)] = jnp.where(cvals > 0, blk, 0.0)

        wb = pl.multiple_of(nlo * dp, 8)
        pltpu.sync_copy(accs, out_s.at[pl.ds(wb, nt * dp)])
        pltpu.sync_copy(accq, out_q.at[pl.ds(wb, nt * dp)])
        pltpu.sync_copy(accm, out_m.at[pl.ds(wb, nt * dp)])
        pltpu.sync_copy(accc, out_c.at[pl.ds(pl.multiple_of(nlo, 8), nt)])

    f32 = jnp.float32
    return pl.kernel(
        body,
        out_type=(
            jax.ShapeDtypeStruct((npad * dp,), f32),
            jax.ShapeDtypeStruct((npad * dp,), f32),
            jax.ShapeDtypeStruct((npad * dp,), f32),
            jax.ShapeDtypeStruct((npad,), f32),
        ),
        mesh=mesh,
        compiler_params=pltpu.CompilerParams(needs_layout_passes=False),
        scratch_types=(
            pltpu.VMEM((16,), jnp.int32),
            pltpu.VMEM((ch,), jnp.int32),
            pltpu.VMEM((ch,), jnp.int32),
            pltpu.VMEM((ch, 128), f32),
            pltpu.VMEM((ch,), jnp.int32),
            pltpu.VMEM((ch,), jnp.int32),
            pltpu.VMEM((ch, 128), f32),
            pltpu.VMEM((nt * dp,), f32),
            pltpu.VMEM((nt * dp,), f32),
            pltpu.VMEM((nt * dp,), f32),
            pltpu.VMEM((nt,), f32),
            pltpu.SemaphoreType.DMA,
            pltpu.SemaphoreType.DMA,
        ),
    )


def _make_gat_kernel(hc, hw, cdim, ch, npasses, nbuf=2):
    nts = NT // npasses
    mesh = plsc.VectorSubcoreMesh(core_axis_name="c", subcore_axis_name="s")

    def body(hp, as8, ad8, cshift, toffs, srcs, dsts, out_acc, out_den,
             offv, cval, idx_v, dst_v, hrows, arows, idx_v1, dst_v1, hrows1,
             arows1, drows, acc, accd, sem, sem1):
        cid = lax.axis_index("c")
        sid = lax.axis_index("s")
        t = sid * 2 + cid
        iota = lax.iota(jnp.int32, 16)
        pltpu.sync_copy(toffs.at[t], offv)
        pltpu.sync_copy(cshift, cval)
        ov = offv[...]
        cv = cval[...]
        zf = jnp.zeros((16,), jnp.float32)

        def pass_body(p, _):
            a0 = _extract(ov, 3 * p, iota)
            s0 = _extract(ov, 3 * p + 1, iota)
            e0 = _extract(ov, 3 * p + 2, iota)
            nlo = t * NT + p * nts

            pltpu.sync_copy(ad8.at[pl.ds(nlo, nts)], drows)

            def init_body(i, _):
                acc[pl.ds(i * 16, 16)] = zf
                return 0
            lax.fori_loop(0, nts * hc // 16, init_body, 0)

            def initd(i, _):
                accd[pl.ds(i * 16, 16)] = zf
                return 0
            lax.fori_loop(0, nts, initd, 0)

            nchunks = (e0 - a0 + ch - 1) // ch
            npairs = (nchunks + 1) // 2
            bufs = ((idx_v, dst_v, hrows, arows, sem),
                    (idx_v1, dst_v1, hrows1, arows1, sem1))

            def issue(b, base):
                iv, dv, hr, ar, sm = bufs[b]
                pltpu.sync_copy(srcs.at[pl.ds(base, ch)], iv)
                pltpu.sync_copy(dsts.at[pl.ds(base, ch)], dv)
                da = pltpu.async_copy(hp.at[iv], hr, sm)
                db = pltpu.async_copy(as8.at[iv], ar, sm)
                return da, db

            def process(b, base):
                iv, dv, hrows_b, arows_b, sm = bufs[b]
                for j in range(ch // 16):
                    dst16 = dv[pl.ds(16 * j, 16)]
                    ge = base + 16 * j + iota
                    vi = jnp.where(jnp.logical_and(ge >= s0, ge < e0), 1, 0)
                    for l in range(16):
                        dsel = _lanebc(dst16, l)
                        vb = _lanebc(vi, l) != 0
                        av = arows_b[16 * j + l, pl.ds(0, 16)]
                        adv = plsc.load_gather(drows, [dsel - nlo, iota], mask=vb)
                        ev = av + adv
                        ev = jnp.where(ev >= 0, ev, 0.2 * ev)
                        exv = jnp.exp(ev - cv)
                        plsc.addupdate_scatter(
                            accd, [(dsel - nlo) * 16 + iota], exv, mask=vb)
                        rowb = (dsel - nlo) * hc
                        for g in range(hc // 16):
                            head = (16 * g) // cdim
                            mult = _lanebc(exv, head)
                            msg = hrows_b[16 * j + l, pl.ds(16 * g, 16)]
                            plsc.addupdate_scatter(
                                acc, [rowb + 16 * g + iota], msg * mult, mask=vb)

            if nbuf == 2:
                def pair_body(cp, _):
                    b0 = pl.multiple_of(a0 + (2 * cp) * ch, 8)
                    b1 = pl.multiple_of(b0 + ch, 8)
                    da0, db0 = issue(0, b0)
                    da1, db1 = issue(1, b1)
                    da0.wait()
                    db0.wait()
                    process(0, b0)
                    da1.wait()
                    db1.wait()
                    process(1, b1)
                    return 0
                lax.fori_loop(0, npairs, pair_body, 0)
            else:
                def chunk_1buf(c, _):
                    b0 = pl.multiple_of(a0 + c * ch, 8)
                    da0, db0 = issue(0, b0)
                    da0.wait()
                    db0.wait()
                    process(0, b0)
                    return 0
                lax.fori_loop(0, nchunks, chunk_1buf, 0)

            pltpu.sync_copy(acc, out_acc.at[pl.ds(pl.multiple_of(nlo * hc, 8), nts * hc)])
            pltpu.sync_copy(accd, out_den.at[pl.ds(pl.multiple_of(nlo * 16, 8), nts * 16)])
            return 0
        lax.fori_loop(0, npasses, pass_body, 0)

    f32 = jnp.float32
    return pl.kernel(
        body,
        out_type=(
            jax.ShapeDtypeStruct((NPAD * hc,), f32),
            jax.ShapeDtypeStruct((NPAD * 16,), f32),
        ),
        mesh=mesh,
        compiler_params=pltpu.CompilerParams(needs_layout_passes=False),
        scratch_types=(
            pltpu.VMEM((16,), jnp.int32),
            pltpu.VMEM((16,), f32),
            pltpu.VMEM((ch,), jnp.int32),
            pltpu.VMEM((ch,), jnp.int32),
            pltpu.VMEM((ch, hw), f32),
            pltpu.VMEM((ch, 128), f32),
            pltpu.VMEM((ch,), jnp.int32),
            pltpu.VMEM((ch,), jnp.int32),
            pltpu.VMEM((ch, hw) if nbuf == 2 else (16, 128), f32),
            pltpu.VMEM((ch, 128), f32),
            pltpu.VMEM((nts, 16), f32),
            pltpu.VMEM((nts * hc,), f32),
            pltpu.VMEM((nts * 16,), f32),
            pltpu.SemaphoreType.DMA,
            pltpu.SemaphoreType.DMA,
        ),
    )


def _ec_mlp_block(s_ref, q_ref, m_ref, c_ref, w1_ref, b1_ref, w2_ref, b2_ref, o_ref):
    cnt = c_ref[...]
    denom = jnp.maximum(cnt, 1.0)
    s = s_ref[...]
    q = q_ref[...]
    m = jnp.where(cnt > 0, m_ref[...], 0.0)
    means = s / denom
    var = jnp.maximum(q / denom - means * means, 0.0)
    aggr = jnp.concatenate([s, m, means, var], axis=1)
    h = jnp.maximum(jnp.dot(aggr, w1_ref[...], preferred_element_type=jnp.float32) + b1_ref[...], 0.0)
    z = jnp.dot(h, w2_ref[...], preferred_element_type=jnp.float32) + b2_ref[...]
    o_ref[...] = jax.nn.sigmoid(z)


def _ec_mlp(s, q, m, cnt, w1p, b1p, w2p, b2p, dp, d2p):
    rb = 1024
    grid = (NPAD // rb,)
    d4 = 4 * dp
    return pl.pallas_call(
        _ec_mlp_block,
        grid=grid,
        in_specs=[
            pl.BlockSpec((rb, dp), lambda i: (i, 0)),
            pl.BlockSpec((rb, dp), lambda i: (i, 0)),
            pl.BlockSpec((rb, dp), lambda i: (i, 0)),
            pl.BlockSpec((rb, 1), lambda i: (i, 0)),
            pl.BlockSpec((d4, d2p), lambda i: (0, 0)),
            pl.BlockSpec((1, d2p), lambda i: (0, 0)),
            pl.BlockSpec((d2p, d2p), lambda i: (0, 0)),
            pl.BlockSpec((1, d2p), lambda i: (0, 0)),
        ],
        out_specs=pl.BlockSpec((rb, d2p), lambda i: (i, 0)),
        out_shape=jax.ShapeDtypeStruct((NPAD, d2p), jnp.float32),
    )(s, q, m, cnt, w1p, b1p, w2p, b2p)


def _gat_proj_block(x_ref, w_ref, as_ref, ad_ref, h_ref, a_ref, d_ref):
    h = jnp.dot(x_ref[...], w_ref[...], preferred_element_type=jnp.float32)
    h_ref[...] = h
    a_ref[...] = jnp.dot(h, as_ref[...], preferred_element_type=jnp.float32)
    d_ref[...] = jnp.dot(h, ad_ref[...], preferred_element_type=jnp.float32)


def _gat_proj(xp, wp, asw, adw):
    k = xp.shape[1]
    hw = wp.shape[1]
    rb = 512
    grid = (NPAD // rb,)
    return pl.pallas_call(
        _gat_proj_block,
        grid=grid,
        in_specs=[
            pl.BlockSpec((rb, k), lambda i: (i, 0)),
            pl.BlockSpec((k, hw), lambda i: (0, 0)),
            pl.BlockSpec((hw, 128), lambda i: (0, 0)),
            pl.BlockSpec((hw, 16), lambda i: (0, 0)),
        ],
        out_specs=[
            pl.BlockSpec((rb, hw), lambda i: (i, 0)),
            pl.BlockSpec((rb, 128), lambda i: (i, 0)),
            pl.BlockSpec((rb, 16), lambda i: (i, 0)),
        ],
        out_shape=[
            jax.ShapeDtypeStruct((NPAD, hw), jnp.float32),
            jax.ShapeDtypeStruct((NPAD, 128), jnp.float32),
            jax.ShapeDtypeStruct((NPAD, 16), jnp.float32),
        ],
    )(xp, wp, asw, adw)


def _gat_fin_block(acc_ref, den_ref, bias_ref, g_ref, b_ref, o_ref):
    z = acc_ref[...] / (den_ref[...] + 1e-16) + bias_ref[...]
    s = jax.nn.sigmoid(z)
    mu = jnp.mean(s, axis=0, keepdims=True)
    var = jnp.mean((s - mu) * (s - mu), axis=0, keepdims=True)
    o_ref[...] = (s - mu) * jax.lax.rsqrt(var + 1e-5) * g_ref[...] + b_ref[...]


def _gat_fin(acc, den_exp, bias, g, b):
    n, hc = acc.shape
    blk = min(hc, 128)
    return pl.pallas_call(
        _gat_fin_block,
        grid=(hc // blk,),
        in_specs=[
            pl.BlockSpec((n, blk), lambda j: (0, j)),
            pl.BlockSpec((n, blk), lambda j: (0, j)),
            pl.BlockSpec((1, blk), lambda j: (0, j)),
            pl.BlockSpec((1, blk), lambda j: (0, j)),
            pl.BlockSpec((1, blk), lambda j: (0, j)),
        ],
        out_specs=pl.BlockSpec((n, blk), lambda j: (0, j)),
        out_shape=jax.ShapeDtypeStruct((n, hc), jnp.float32),
    )(acc, den_exp, bias, g, b)


def _head_block(flat_ref, w1_ref, b1_ref, w2_ref, b2_ref, o_ref):
    f1 = jnp.dot(flat_ref[...], w1_ref[...], preferred_element_type=jnp.float32) + b1_ref[...]
    f1 = jnp.where(f1 > 0, f1, jnp.exp(jnp.minimum(f1, 0.0)) - 1.0)
    o_ref[...] = jnp.dot(f1, w2_ref[...], preferred_element_type=jnp.float32) + b2_ref[...]


def _head(flat, w1, b1, w2, b2):
    return pl.pallas_call(
        _head_block,
        out_shape=jax.ShapeDtypeStruct((flat.shape[0], w2.shape[1]), jnp.float32),
    )(flat, w1, b1, w2, b2)


def _pad_ec_w1(w1, d, dp, d2, d2p):
    w1p = jnp.zeros((4 * dp, d2p), jnp.float32)
    for k in range(4):
        w1p = w1p.at[k * dp:k * dp + d, :d2].set(w1[k * d:(k + 1) * d, :])
    return w1p


def _pad_mat(w, r, c):
    return jnp.zeros((r, c), jnp.float32).at[:w.shape[0], :w.shape[1]].set(w)


def _attn_mat(a, heads, cdim, hw, cols):
    m = jnp.zeros((hw, cols), jnp.float32)
    for h in range(heads):
        m = m.at[h * cdim:(h + 1) * cdim, h].set(a[h])
    return m


def _event_layer(xp, dp, d2, d2p, w1, b1, w2, b2, stats_fn, sort_args):
    srcs, dsts, toffs = sort_args
    s, q, m, c = stats_fn(xp, toffs, srcs, dsts)
    s = s.reshape(NPAD, dp)
    q = q.reshape(NPAD, dp)
    m = m.reshape(NPAD, dp)
    c = c.reshape(NPAD, 1)
    w1p = _pad_ec_w1(w1, w1.shape[0] // 4, dp, d2, d2p)
    b1p = _pad_mat(b1.reshape(1, -1), 1, d2p)
    w2p = _pad_mat(w2, d2p, d2p)
    b2p = _pad_mat(b2.reshape(1, -1), 1, d2p)
    return _ec_mlp(s, q, m, c, w1p, b1p, w2p, b2p, dp, d2p)


def _gat_layer(xin, w, a_s, a_d, bias, bn_g, bn_b, heads, cdim, gk, sort_args, kdim, hw):
    srcs, dsts, toffs = sort_args
    hc = heads * cdim
    xp = _pad_mat(xin, NPAD, kdim)
    wp = _pad_mat(w, kdim, hw)
    asw = _attn_mat(a_s, heads, cdim, hw, 128)
    adw = _attn_mat(a_d, heads, cdim, hw, 16)
    h, as8, ad8 = _gat_proj(xp, wp, asw, adw)
    cshift = jnp.maximum(jnp.max(as8) + jnp.max(ad8), 0.0)
    cshift16 = jnp.full((16,), cshift, jnp.float32)
    acc, den = gk(h, as8, ad8, cshift16, toffs, srcs, dsts)
    acc = acc.reshape(NPAD, hc)[:N]
    den8 = den.reshape(NPAD, 16)[:N, :heads]
    den_exp = jnp.repeat(den8, cdim, axis=1)
    return _gat_fin(acc, den_exp, bias.reshape(1, hc), bn_g.reshape(1, hc), bn_b.reshape(1, hc))


def kernel(x, edge_index, pos, batch,
           ec1_W1, ec1_b1, ec1_W2, ec1_b2,
           ec2_W1, ec2_b1, ec2_W2, ec2_b2,
           ec3_W1, ec3_b1, ec3_W2, ec3_b2,
           g0_W, g0_as, g0_ad, g0_b,
           g1_W, g1_as, g1_ad, g1_b,
           g2_W, g2_as, g2_ad, g2_b,
           bn0_g, bn0_b, bn1_g, bn1_b, bn2_g, bn2_b,
           fc1_W, fc1_b, fc2_W, fc2_b):
    src = edge_index[0].astype(jnp.int32)
    dst = edge_index[1].astype(jnp.int32)

    perm = jnp.argsort(dst)
    src_s = src[perm]
    dst_s = dst[perm]
    EPAD = E + 256
    srcs = jnp.zeros((EPAD,), jnp.int32).at[:E].set(src_s)
    dsts = jnp.full((EPAD,), N, jnp.int32).at[:E].set(dst_s)

    bounds80 = jnp.arange(129, dtype=jnp.int32) * 80
    offs80 = jnp.searchsorted(dst_s, bounds80).astype(jnp.int32)
    offs320 = offs80[::4]
    a320 = (offs320[:32] // 8) * 8
    toffs1 = jnp.zeros((NTILES, 16), jnp.int32)
    toffs1 = toffs1.at[:, 0].set(a320)
    toffs1 = toffs1.at[:, 1].set(offs320[:32])
    toffs1 = toffs1.at[:, 2].set(offs320[1:33])
    a80 = (offs80[:128] // 8) * 8
    toffs2 = jnp.zeros((NTILES, 16), jnp.int32)
    for p in range(4):
        toffs2 = toffs2.at[:, 3 * p].set(a80[p::4])
        toffs2 = toffs2.at[:, 3 * p + 1].set(offs80[:128][p::4])
        toffs2 = toffs2.at[:, 3 * p + 2].set(offs80[1:][p::4])
    sa1 = (srcs, dsts, toffs1)

    stats16 = _make_stats_kernel(16, 128, NPAD, NT, False)
    stats64 = _make_stats_kernel(64, 32, NPAD, NT, False)
    stats80 = _make_stats_kernel(80, 32, NPAD, NT, False)
    gat512 = _make_gat_kernel(512, 512, 64, 16, 4)
    gat128 = _make_gat_kernel(128, 128, 16, 32, 1)
    gat16 = _make_gat_kernel(16, 128, 16, 64, 1)

    xp1 = _pad_mat(x, N, 128)
    h1 = _event_layer(xp1, 16, 12, 16, ec1_W1, ec1_b1, ec1_W2, ec1_b2, stats16, sa1)
    xp2 = _pad_mat(jnp.concatenate([x, h1[:N, :12]], axis=1), N, 128)
    h2 = _event_layer(xp2, 64, 60, 64, ec2_W1, ec2_b1, ec2_W2, ec2_b2, stats64, sa1)
    xp3 = _pad_mat(jnp.concatenate([xp2[:, :15], h2[:N, :60]], axis=1), N, 128)
    h3 = _event_layer(xp3, 80, 300, 320, ec3_W1, ec3_b1, ec3_W2, ec3_b2, stats80, sa1)
    x4 = jnp.concatenate([xp3[:, :75], h3[:N, :300]], axis=1)

    g0 = _gat_layer(x4, g0_W, g0_as, g0_ad, g0_b, bn0_g, bn0_b, 8, 64, gat512,
                    (srcs, dsts, toffs2), 384, 512)
    g1 = _gat_layer(g0, g1_W, g1_as, g1_ad, g1_b, bn1_g, bn1_b, 8, 16, gat128,
                    sa1, 512, 128)
    g2 = _gat_layer(g1, g2_W, g2_as, g2_ad, g2_b, bn2_g, bn2_b, 1, 16, gat16,
                    sa1, 128, 128)

    vox = jnp.clip(jnp.floor(pos / 0.25).astype(jnp.int32), 0, 3)
    cid = vox[:, 0] * 16 + vox[:, 1] * 4 + vox[:, 2]
    gid = batch.astype(jnp.int32) * 64 + cid
    permg = jnp.argsort(gid).astype(jnp.int32)
    gid_s = gid[permg]
    EPG = 10240
    srcg = jnp.zeros((EPG,), jnp.int32).at[:N].set(permg)
    dstg = jnp.full((EPG,), 1024, jnp.int32).at[:N].set(gid_s)
    boundsg = jnp.arange(33, dtype=jnp.int32) * 32
    offsg = jnp.searchsorted(gid_s, boundsg).astype(jnp.int32)
    ag = (offsg[:32] // 8) * 8
    toffsg = jnp.zeros((NTILES, 16), jnp.int32)
    toffsg = toffsg.at[:, 0].set(ag)
    toffsg = toffsg.at[:, 1].set(offsg[:32])
    toffsg = toffsg.at[:, 2].set(offsg[1:33])
    statsg = _make_stats_kernel(16, 128, 1024, 32, True)
    g2p = _pad_mat(g2, N, 128)
    _, _, pm, _ = statsg(g2p, toffsg, srcg, dstg)
    pooled = pm.reshape(1024, 16)
    flat = pooled.reshape(B, 1024)
    return _head(flat, fc1_W, fc1_b, fc2_W, fc2_b)
